# Initial kernel scaffold; baseline (speedup 1.0000x reference)
import jax
import jax.numpy as jnp
from jax.experimental import pallas as pl


def kernel(obj_dists1, obj_feats, obj_labels, box_priors, boxes_per_cls, W, b):
    raise NotImplementedError("write your pallas kernel here")


# speedup vs baseline: 18.2671x; 18.2671x over previous
import jax
import jax.numpy as jnp
from jax.experimental import pallas as pl
from jax.experimental.pallas import tpu as pltpu

N = 1000
C = 51
H = 256
BIG = 2**30


def _nms_kernel(d1_ref, feats_ref, wt_ref, b_ref,
                x1t_ref, y1t_ref, x2t_ref, y2t_ref,
                d2_ref, preds_ref,
                pt_ref, flat_ref, areas_ref):
    d2 = jnp.dot(feats_ref[...], wt_ref[...],
                 preferred_element_type=jnp.float32)
    d2 = d2 + b_ref[...] + d1_ref[...]
    d2_ref[...] = d2

    d2t = jnp.transpose(d2)
    mx = jnp.max(d2t, axis=0, keepdims=True)
    e = jnp.exp(d2t - mx)
    s = jnp.sum(e, axis=0, keepdims=True)
    pt = e / s
    row = jax.lax.broadcasted_iota(jnp.int32, (C, N), 0)
    col = jax.lax.broadcasted_iota(jnp.int32, (C, N), 1)
    pt = jnp.where(row == 0, 0.0, pt)
    pt_ref[...] = pt
    flat_ref[...] = col * C + row
    areas_ref[...] = ((x2t_ref[...] - x1t_ref[...]) + 1.0) * \
                     ((y2t_ref[...] - y1t_ref[...]) + 1.0)

    lane = jax.lax.broadcasted_iota(jnp.int32, (1, N), 1)

    def body(_, preds):
        p = pt_ref[...]
        m = jnp.max(p)
        flat = jnp.min(jnp.where(p == m, flat_ref[...], BIG))
        n_ = flat // C
        c_ = flat % C
        x1r = x1t_ref[pl.ds(c_, 1), :]
        y1r = y1t_ref[pl.ds(c_, 1), :]
        x2r = x2t_ref[pl.ds(c_, 1), :]
        y2r = y2t_ref[pl.ds(c_, 1), :]
        ar = areas_ref[pl.ds(c_, 1), :]
        sel = lane == n_
        X1 = jnp.sum(jnp.where(sel, x1r, 0.0))
        Y1 = jnp.sum(jnp.where(sel, y1r, 0.0))
        X2 = jnp.sum(jnp.where(sel, x2r, 0.0))
        Y2 = jnp.sum(jnp.where(sel, y2r, 0.0))
        A = jnp.sum(jnp.where(sel, ar, 0.0))
        iw = jnp.maximum(jnp.minimum(X2, x2r) - jnp.maximum(X1, x1r) + 1.0, 0.0)
        ih = jnp.maximum(jnp.minimum(Y2, y2r) - jnp.maximum(Y1, y1r) + 1.0, 0.0)
        inters = iw * ih
        union = (-inters + ar) + A
        ovl = (inters / union) >= 0.5
        prow = pt_ref[pl.ds(c_, 1), :]
        pt_ref[pl.ds(c_, 1), :] = jnp.where(ovl, 0.0, prow)
        pcur = pt_ref[...]
        pt_ref[...] = jnp.where(col == n_, -1.0, pcur)
        return jnp.where(sel, c_, preds)

    preds = jax.lax.fori_loop(0, N, body,
                              jnp.zeros((1, N), dtype=jnp.int32))
    preds_ref[...] = preds


def kernel(obj_dists1, obj_feats, obj_labels, box_priors, boxes_per_cls, W, b):
    del obj_labels, box_priors
    wt = W.T
    b2 = b[None, :]
    x1t = boxes_per_cls[:, :, 0].T
    y1t = boxes_per_cls[:, :, 1].T
    x2t = boxes_per_cls[:, :, 2].T
    y2t = boxes_per_cls[:, :, 3].T
    d2, preds = pl.pallas_call(
        _nms_kernel,
        out_shape=(
            jax.ShapeDtypeStruct((N, C), jnp.float32),
            jax.ShapeDtypeStruct((1, N), jnp.int32),
        ),
        scratch_shapes=[
            pltpu.VMEM((C, N), jnp.float32),
            pltpu.VMEM((C, N), jnp.int32),
            pltpu.VMEM((C, N), jnp.float32),
        ],
    )(obj_dists1, obj_feats, wt, b2, x1t, y1t, x2t, y2t)
    return d2, preds[0]
